# baseline (speedup 1.0000x reference)
import functools

import jax
import jax.numpy as jnp
from jax import lax
from jax.experimental import pallas as pl
from jax.experimental.pallas import tpu as pltpu
from jax.experimental.pallas import tpu_sc as plsc

N = 100000
H = 128
M = 1024
K = 3


_BLK = 4000


def _proj_body(v_ref, w_ref, o_ref):
    o_ref[...] = lax.dot_general(
        v_ref[...], w_ref[...],
        (((1,), (0,)), ((), ())),
        preferred_element_type=jnp.float32,
    )


def _tc_project(v2, wexp):
    return pl.pallas_call(
        _proj_body,
        grid=(N // _BLK,),
        in_specs=[
            pl.BlockSpec((_BLK, H * K), lambda i: (i, 0)),
            pl.BlockSpec((H * K, K), lambda i: (0, 0)),
        ],
        out_specs=pl.BlockSpec((_BLK, K), lambda i: (i, 0)),
        out_shape=jax.ShapeDtypeStruct((N, K), jnp.float32),
    )(v2, wexp)


_NW = 32
_CH = 3200
_NPAD = _NW * _CH
_CF = _CH * K
_ACC = M * K
_SL = _ACC // 16


def _sc_mesh():
    return plsc.VectorSubcoreMesh(core_axis_name="c", subcore_axis_name="s")


def _segsum_body(vals_hbm, ids_hbm, out_hbm, vals_v, ids_v, acc_v, sum_v,
                 tmp_v, shared):
    c = lax.axis_index("c")
    s = lax.axis_index("s")
    wid = c * 16 + s

    pltpu.sync_copy(vals_hbm.at[pl.ds(wid * _CF, _CF)], vals_v)
    pltpu.sync_copy(ids_hbm.at[pl.ds(wid * _CH, _CH)], ids_v)

    z16 = jnp.zeros((16,), jnp.float32)

    def _zero(j, _):
        acc_v[pl.ds(j * 16, 16)] = z16
        return 0

    lax.fori_loop(0, _ACC // 16, _zero, 0)

    iota16 = lax.iota(jnp.int32, 16)

    def _accum(g, _):
        o = g * 16
        fi = o + iota16
        ai = lax.shift_right_logical(fi * 21846, 16)
        kk = fi - ai * 3
        idv = plsc.load_gather(ids_v, [ai])
        vals = vals_v[pl.ds(o, 16)]
        plsc.addupdate_scatter(acc_v, [idv * 3 + kk], vals)
        return 0

    lax.fori_loop(0, _CF // 16, _accum, 0)

    pltpu.sync_copy(acc_v, shared.at[pl.ds(s * _ACC, _ACC)])
    plsc.subcore_barrier()

    def _zero2(j, _):
        sum_v[pl.ds(j * 16, 16)] = z16
        return 0

    lax.fori_loop(0, _SL // 16, _zero2, 0)

    def _red(j, _):
        pltpu.sync_copy(shared.at[pl.ds(j * _ACC + s * _SL, _SL)], tmp_v)

        def _add(t, _):
            sum_v[pl.ds(t * 16, 16)] = (
                sum_v[pl.ds(t * 16, 16)] + tmp_v[pl.ds(t * 16, 16)])
            return 0

        lax.fori_loop(0, _SL // 16, _add, 0)
        return 0

    lax.fori_loop(0, 16, _red, 0)

    pltpu.sync_copy(sum_v, out_hbm.at[pl.ds(c * _ACC + s * _SL, _SL)])


@functools.partial(jax.jit, static_argnums=())
def _sc_segsum(vals_flat, ids):
    f = pl.kernel(
        _segsum_body,
        mesh=_sc_mesh(),
        out_type=jax.ShapeDtypeStruct((2 * _ACC,), jnp.float32),
        compiler_params=pltpu.CompilerParams(needs_layout_passes=False),
        scratch_types=[
            pltpu.VMEM((_CF,), jnp.float32),
            pltpu.VMEM((_CH,), jnp.int32),
            pltpu.VMEM((_ACC,), jnp.float32),
            pltpu.VMEM((_SL,), jnp.float32),
            pltpu.VMEM((_SL,), jnp.float32),
            pltpu.VMEM_SHARED((16 * _ACC,), jnp.float32),
        ],
    )
    return f(vals_flat, ids)


def kernel(v, batch, W):
    w = jnp.squeeze(W, axis=0)
    wexp = jnp.kron(w[:, None], jnp.eye(K, dtype=jnp.float32))
    v2 = v.reshape(N, H * K)
    mu_atom = _tc_project(v2, wexp)

    vals_flat = jnp.pad(mu_atom.reshape(-1), (0, _NPAD * K - N * K))
    ids = jnp.pad(batch.astype(jnp.int32), (0, _NPAD - N))
    out2 = _sc_segsum(vals_flat, ids).reshape(2, _ACC)
    return (out2[0] + out2[1]).reshape(M, K)


# speedup vs baseline: 1.0619x; 1.0619x over previous
import functools

import jax
import jax.numpy as jnp
from jax import lax
from jax.experimental import pallas as pl
from jax.experimental.pallas import tpu as pltpu
from jax.experimental.pallas import tpu_sc as plsc

N = 100000
H = 128
M = 1024
K = 3

_NW = 32
_CH = 3200
_NPAD = _NW * _CH
_ACC = M * K
_SL = _ACC // 16
_TAIL = N - (_NW - 1) * _CH


def _proj_body(v_ref, w_ref, o_ref):
    res = lax.dot_general(
        w_ref[...], v_ref[...],
        (((0,), (1,)), ((), ())),
        preferred_element_type=jnp.float32,
    )

    i = pl.program_id(0)

    @pl.when(i < _NW - 1)
    def _():
        o_ref[...] = res

    @pl.when(i == _NW - 1)
    def _():
        col = lax.broadcasted_iota(jnp.int32, (K, _CH), 1)
        o_ref[...] = jnp.where(col < _TAIL, res, 0.0)


def _tc_project(v2, wexp):
    return pl.pallas_call(
        _proj_body,
        grid=(_NW,),
        in_specs=[
            pl.BlockSpec((_CH, H * K), lambda i: (i, 0)),
            pl.BlockSpec((H * K, K), lambda i: (0, 0)),
        ],
        out_specs=pl.BlockSpec((K, _CH), lambda i: (0, i)),
        out_shape=jax.ShapeDtypeStruct((K, _NPAD), jnp.float32),
    )(v2, wexp)


def _segsum_body(vals_hbm, ids_hbm, out_hbm, vals_v, ids_v, acc_v, sum_v,
                 tmp_v, shared):
    c = lax.axis_index("c")
    s = lax.axis_index("s")
    wid = c * 16 + s
    base = wid * _CH

    for k in range(K):
        pltpu.sync_copy(vals_hbm.at[pl.ds(k * _NPAD + base, _CH)],
                        vals_v.at[pl.ds(k * _CH, _CH)])

    @pl.when(wid < _NW - 1)
    def _():
        pltpu.sync_copy(ids_hbm.at[pl.ds(base, _CH)], ids_v)

    z16i = jnp.zeros((16,), jnp.int32)

    @pl.when(wid == _NW - 1)
    def _():
        pltpu.sync_copy(ids_hbm.at[pl.ds((_NW - 1) * _CH, _TAIL)],
                        ids_v.at[pl.ds(0, _TAIL)])

        def _zpad(j, _):
            ids_v[pl.ds(_TAIL + j * 16, 16)] = z16i
            return 0

        lax.fori_loop(0, (_CH - _TAIL) // 16, _zpad, 0)

    z16 = jnp.zeros((16,), jnp.float32)

    def _zero(j, _):
        acc_v[pl.ds(j * 16, 16)] = z16
        return 0

    lax.fori_loop(0, _ACC // 16, _zero, 0)

    for k in range(K):
        def _accum(g, _, k=k):
            o = g * 16
            idv = ids_v[pl.ds(o, 16)]
            vals = vals_v[pl.ds(k * _CH + o, 16)]
            plsc.addupdate_scatter(acc_v, [idv * 3 + k], vals)
            return 0

        lax.fori_loop(0, _CH // 16, _accum, 0)

    pltpu.sync_copy(acc_v, shared.at[pl.ds(s * _ACC, _ACC)])
    plsc.subcore_barrier()

    def _zero2(j, _):
        sum_v[pl.ds(j * 16, 16)] = z16
        return 0

    lax.fori_loop(0, _SL // 16, _zero2, 0)

    def _red(j, _):
        pltpu.sync_copy(shared.at[pl.ds(j * _ACC + s * _SL, _SL)], tmp_v)

        def _add(t, _):
            sum_v[pl.ds(t * 16, 16)] = (
                sum_v[pl.ds(t * 16, 16)] + tmp_v[pl.ds(t * 16, 16)])
            return 0

        lax.fori_loop(0, _SL // 16, _add, 0)
        return 0

    lax.fori_loop(0, 16, _red, 0)

    pltpu.sync_copy(sum_v, out_hbm.at[pl.ds(c * _ACC + s * _SL, _SL)])


def _sc_segsum(vals_flat, ids):
    f = pl.kernel(
        _segsum_body,
        mesh=plsc.VectorSubcoreMesh(core_axis_name="c", subcore_axis_name="s"),
        out_type=jax.ShapeDtypeStruct((2 * _ACC,), jnp.float32),
        compiler_params=pltpu.CompilerParams(needs_layout_passes=False),
        scratch_types=[
            pltpu.VMEM((K * _CH,), jnp.float32),
            pltpu.VMEM((_CH,), jnp.int32),
            pltpu.VMEM((_ACC,), jnp.float32),
            pltpu.VMEM((_SL,), jnp.float32),
            pltpu.VMEM((_SL,), jnp.float32),
            pltpu.VMEM_SHARED((16 * _ACC,), jnp.float32),
        ],
    )
    return f(vals_flat, ids)


def kernel(v, batch, W):
    w = jnp.squeeze(W, axis=0)
    wexp = jnp.kron(w[:, None], jnp.eye(K, dtype=jnp.float32))
    v2 = v.reshape(N, H * K)
    mu_t = _tc_project(v2, wexp)
    out2 = _sc_segsum(mu_t.reshape(-1), batch.astype(jnp.int32))
    out2 = out2.reshape(2, _ACC)
    return (out2[0] + out2[1]).reshape(M, K)


# speedup vs baseline: 4.4895x; 4.2277x over previous
import functools

import jax
import jax.numpy as jnp
from jax import lax
from jax.experimental import pallas as pl
from jax.experimental.pallas import tpu as pltpu
from jax.experimental.pallas import tpu_sc as plsc

N = 100000
H = 128
M = 1024
K = 3

_NW = 32
_CH = 3200
_NPAD = _NW * _CH
_ACC = M * K
_SL = _ACC // 16
_TAIL = N - (_NW - 1) * _CH


_BN = 4096
_NB = _NPAD // _BN
_TCTAIL = N - (_NB - 1) * _BN


def _proj_body(v_ref, w_ref, o_ref):
    wb = jnp.broadcast_to(w_ref[...], (K, H))
    res = lax.dot_general(
        v_ref[...], wb,
        (((2,), (1,)), ((0,), (0,))),
        preferred_element_type=jnp.float32,
    )

    i = pl.program_id(0)

    @pl.when(i < _NB - 1)
    def _():
        o_ref[...] = res

    @pl.when(i == _NB - 1)
    def _():
        col = lax.broadcasted_iota(jnp.int32, (K, _BN), 1)
        o_ref[...] = jnp.where(col < _TCTAIL, res, 0.0)


def _tc_project(vt, w):
    return pl.pallas_call(
        _proj_body,
        grid=(_NB,),
        in_specs=[
            pl.BlockSpec((K, _BN, H), lambda i: (0, i, 0)),
            pl.BlockSpec((1, H), lambda i: (0, 0)),
        ],
        out_specs=pl.BlockSpec((K, _BN), lambda i: (0, i)),
        out_shape=jax.ShapeDtypeStruct((K, _NPAD), jnp.float32),
    )(vt, w)


def _segsum_body(vals_hbm, ids_hbm, out_hbm, vals_v, ids_v, acc_v, sum_v,
                 tmp_v, shared):
    c = lax.axis_index("c")
    s = lax.axis_index("s")
    wid = c * 16 + s
    base = wid * _CH

    for k in range(K):
        pltpu.sync_copy(vals_hbm.at[pl.ds(k * _NPAD + base, _CH)],
                        vals_v.at[pl.ds(k * _CH, _CH)])

    @pl.when(wid < _NW - 1)
    def _():
        pltpu.sync_copy(ids_hbm.at[pl.ds(base, _CH)], ids_v)

    z16i = jnp.zeros((16,), jnp.int32)

    @pl.when(wid == _NW - 1)
    def _():
        pltpu.sync_copy(ids_hbm.at[pl.ds((_NW - 1) * _CH, _TAIL)],
                        ids_v.at[pl.ds(0, _TAIL)])

        def _zpad(j, _):
            ids_v[pl.ds(_TAIL + j * 16, 16)] = z16i
            return 0

        lax.fori_loop(0, (_CH - _TAIL) // 16, _zpad, 0)

    z16 = jnp.zeros((16,), jnp.float32)

    def _zero(j, _):
        acc_v[pl.ds(j * 16, 16)] = z16
        return 0

    lax.fori_loop(0, _ACC // 16, _zero, 0)

    for k in range(K):
        def _accum(g, _, k=k):
            o = g * 16
            idv = ids_v[pl.ds(o, 16)]
            vals = vals_v[pl.ds(k * _CH + o, 16)]
            plsc.addupdate_scatter(acc_v, [idv * 3 + k], vals)
            return 0

        lax.fori_loop(0, _CH // 16, _accum, 0)

    pltpu.sync_copy(acc_v, shared.at[pl.ds(s * _ACC, _ACC)])
    plsc.subcore_barrier()

    def _zero2(j, _):
        sum_v[pl.ds(j * 16, 16)] = z16
        return 0

    lax.fori_loop(0, _SL // 16, _zero2, 0)

    def _red(j, _):
        pltpu.sync_copy(shared.at[pl.ds(j * _ACC + s * _SL, _SL)], tmp_v)

        def _add(t, _):
            sum_v[pl.ds(t * 16, 16)] = (
                sum_v[pl.ds(t * 16, 16)] + tmp_v[pl.ds(t * 16, 16)])
            return 0

        lax.fori_loop(0, _SL // 16, _add, 0)
        return 0

    lax.fori_loop(0, 16, _red, 0)

    pltpu.sync_copy(sum_v, out_hbm.at[pl.ds(c * _ACC + s * _SL, _SL)])


def _sc_segsum(vals_flat, ids):
    f = pl.kernel(
        _segsum_body,
        mesh=plsc.VectorSubcoreMesh(core_axis_name="c", subcore_axis_name="s"),
        out_type=jax.ShapeDtypeStruct((2 * _ACC,), jnp.float32),
        compiler_params=pltpu.CompilerParams(needs_layout_passes=False),
        scratch_types=[
            pltpu.VMEM((K * _CH,), jnp.float32),
            pltpu.VMEM((_CH,), jnp.int32),
            pltpu.VMEM((_ACC,), jnp.float32),
            pltpu.VMEM((_SL,), jnp.float32),
            pltpu.VMEM((_SL,), jnp.float32),
            pltpu.VMEM_SHARED((16 * _ACC,), jnp.float32),
        ],
    )
    return f(vals_flat, ids)


def kernel(v, batch, W):
    vt = jnp.transpose(v, (2, 0, 1))
    mu_t = _tc_project(vt, W)
    out2 = _sc_segsum(mu_t.reshape(-1), batch.astype(jnp.int32))
    out2 = out2.reshape(2, _ACC)
    return (out2[0] + out2[1]).reshape(M, K)


# speedup vs baseline: 6.1128x; 1.3616x over previous
import functools

import jax
import jax.numpy as jnp
from jax import lax
from jax.experimental import pallas as pl
from jax.experimental.pallas import tpu as pltpu
from jax.experimental.pallas import tpu_sc as plsc

N = 100000
H = 128
M = 1024
K = 3

_NW = 32
_CH = 3200
_NPAD = _NW * _CH
_ACC = M * K
_SL = _ACC // 16
_TAIL = N - (_NW - 1) * _CH


_BN = 4096
_NB = _NPAD // _BN
_TCTAIL = N - (_NB - 1) * _BN


def _proj_body(v_ref, w_ref, o_ref):
    i = pl.program_id(0)
    for k in range(K):
        yk = lax.dot_general(
            w_ref[...], v_ref[k],
            (((1,), (1,)), ((), ())),
            preferred_element_type=jnp.float32,
        )

        @pl.when(i < _NB - 1)
        def _(yk=yk, k=k):
            o_ref[k:k + 1, :] = yk

        @pl.when(i == _NB - 1)
        def _(yk=yk, k=k):
            col = lax.broadcasted_iota(jnp.int32, (1, _BN), 1)
            o_ref[k:k + 1, :] = jnp.where(col < _TCTAIL, yk, 0.0)


def _tc_project(vt, w):
    return pl.pallas_call(
        _proj_body,
        grid=(_NB,),
        in_specs=[
            pl.BlockSpec((K, _BN, H), lambda i: (0, i, 0)),
            pl.BlockSpec((1, H), lambda i: (0, 0)),
        ],
        out_specs=pl.BlockSpec((K, _BN), lambda i: (0, i)),
        out_shape=jax.ShapeDtypeStruct((K, _NPAD), jnp.float32),
    )(vt, w)


def _segsum_body(vals_hbm, ids_hbm, out_hbm, vals_v, ids_v, acc_v, sum_v,
                 tmp_v, shared):
    c = lax.axis_index("c")
    s = lax.axis_index("s")
    wid = c * 16 + s
    base = wid * _CH

    for k in range(K):
        pltpu.sync_copy(vals_hbm.at[pl.ds(k * _NPAD + base, _CH)],
                        vals_v.at[pl.ds(k * _CH, _CH)])

    @pl.when(wid < _NW - 1)
    def _():
        pltpu.sync_copy(ids_hbm.at[pl.ds(base, _CH)], ids_v)

    z16i = jnp.zeros((16,), jnp.int32)

    @pl.when(wid == _NW - 1)
    def _():
        pltpu.sync_copy(ids_hbm.at[pl.ds((_NW - 1) * _CH, _TAIL)],
                        ids_v.at[pl.ds(0, _TAIL)])

        def _zpad(j, _):
            ids_v[pl.ds(_TAIL + j * 16, 16)] = z16i
            return 0

        lax.fori_loop(0, (_CH - _TAIL) // 16, _zpad, 0)

    z16 = jnp.zeros((16,), jnp.float32)

    def _zero(j, _):
        acc_v[pl.ds(j * 16, 16)] = z16
        return 0

    lax.fori_loop(0, _ACC // 16, _zero, 0)

    for k in range(K):
        def _accum(g, _, k=k):
            o = g * 16
            idv = ids_v[pl.ds(o, 16)]
            vals = vals_v[pl.ds(k * _CH + o, 16)]
            plsc.addupdate_scatter(acc_v, [idv * 3 + k], vals)
            return 0

        lax.fori_loop(0, _CH // 16, _accum, 0)

    pltpu.sync_copy(acc_v, shared.at[pl.ds(s * _ACC, _ACC)])
    plsc.subcore_barrier()

    def _zero2(j, _):
        sum_v[pl.ds(j * 16, 16)] = z16
        return 0

    lax.fori_loop(0, _SL // 16, _zero2, 0)

    def _red(j, _):
        pltpu.sync_copy(shared.at[pl.ds(j * _ACC + s * _SL, _SL)], tmp_v)

        def _add(t, _):
            sum_v[pl.ds(t * 16, 16)] = (
                sum_v[pl.ds(t * 16, 16)] + tmp_v[pl.ds(t * 16, 16)])
            return 0

        lax.fori_loop(0, _SL // 16, _add, 0)
        return 0

    lax.fori_loop(0, 16, _red, 0)

    pltpu.sync_copy(sum_v, out_hbm.at[pl.ds(c * _ACC + s * _SL, _SL)])


def _sc_segsum(vals_flat, ids):
    f = pl.kernel(
        _segsum_body,
        mesh=plsc.VectorSubcoreMesh(core_axis_name="c", subcore_axis_name="s"),
        out_type=jax.ShapeDtypeStruct((2 * _ACC,), jnp.float32),
        compiler_params=pltpu.CompilerParams(needs_layout_passes=False),
        scratch_types=[
            pltpu.VMEM((K * _CH,), jnp.float32),
            pltpu.VMEM((_CH,), jnp.int32),
            pltpu.VMEM((_ACC,), jnp.float32),
            pltpu.VMEM((_SL,), jnp.float32),
            pltpu.VMEM((_SL,), jnp.float32),
            pltpu.VMEM_SHARED((16 * _ACC,), jnp.float32),
        ],
    )
    return f(vals_flat, ids)


def kernel(v, batch, W):
    vt = jnp.transpose(v, (2, 0, 1))
    mu_t = _tc_project(vt, W)
    out2 = _sc_segsum(mu_t.reshape(-1), batch.astype(jnp.int32))
    out2 = out2.reshape(2, _ACC)
    return (out2[0] + out2[1]).reshape(M, K)


# speedup vs baseline: 6.4462x; 1.0545x over previous
import functools

import jax
import jax.numpy as jnp
from jax import lax
from jax.experimental import pallas as pl
from jax.experimental.pallas import tpu as pltpu
from jax.experimental.pallas import tpu_sc as plsc

N = 100000
H = 128
M = 1024
K = 3

_NW = 32
_CH = 3200
_NPAD = _NW * _CH
_ACC = M * K
_SL = _ACC // 16
_TAIL = N - (_NW - 1) * _CH


_BN = 6400
_NB = _NPAD // _BN
_TCTAIL = N - (_NB - 1) * _BN


def _proj_body(v_ref, w_ref, o_ref):
    i = pl.program_id(0)
    for k in range(K):
        yk = lax.dot_general(
            w_ref[...], v_ref[k],
            (((1,), (1,)), ((), ())),
            preferred_element_type=jnp.float32,
        )

        @pl.when(i < _NB - 1)
        def _(yk=yk, k=k):
            o_ref[k:k + 1, :] = yk

        @pl.when(i == _NB - 1)
        def _(yk=yk, k=k):
            col = lax.broadcasted_iota(jnp.int32, (1, _BN), 1)
            o_ref[k:k + 1, :] = jnp.where(col < _TCTAIL, yk, 0.0)


def _tc_project(vt, w):
    return pl.pallas_call(
        _proj_body,
        grid=(_NB,),
        in_specs=[
            pl.BlockSpec((K, _BN, H), lambda i: (0, i, 0)),
            pl.BlockSpec((1, H), lambda i: (0, 0)),
        ],
        out_specs=pl.BlockSpec((K, _BN), lambda i: (0, i)),
        out_shape=jax.ShapeDtypeStruct((K, _NPAD), jnp.float32),
    )(vt, w)


def _segsum_body(vals_hbm, ids_hbm, out_hbm, vals_v, ids_v, acc_v, sum_v,
                 tmp_v, shared):
    c = lax.axis_index("c")
    s = lax.axis_index("s")
    wid = c * 16 + s
    base = wid * _CH

    for k in range(K):
        pltpu.sync_copy(vals_hbm.at[pl.ds(k * _NPAD + base, _CH)],
                        vals_v.at[pl.ds(k * _CH, _CH)])

    @pl.when(wid < _NW - 1)
    def _():
        pltpu.sync_copy(ids_hbm.at[pl.ds(base, _CH)], ids_v)

    z16i = jnp.zeros((16,), jnp.int32)

    @pl.when(wid == _NW - 1)
    def _():
        pltpu.sync_copy(ids_hbm.at[pl.ds((_NW - 1) * _CH, _TAIL)],
                        ids_v.at[pl.ds(0, _TAIL)])

        def _zpad(j, _):
            ids_v[pl.ds(_TAIL + j * 16, 16)] = z16i
            return 0

        lax.fori_loop(0, (_CH - _TAIL) // 16, _zpad, 0)

    z16 = jnp.zeros((16,), jnp.float32)

    def _zero(j, _):
        acc_v[pl.ds(j * 16, 16)] = z16
        return 0

    lax.fori_loop(0, _ACC // 16, _zero, 0)

    for k in range(K):
        def _accum(g, _, k=k):
            for u in range(4):
                o = g * 64 + u * 16
                idv = ids_v[pl.ds(o, 16)]
                vals = vals_v[pl.ds(k * _CH + o, 16)]
                plsc.addupdate_scatter(acc_v, [idv * 3 + k], vals)
            return 0

        lax.fori_loop(0, _CH // 64, _accum, 0)

    pltpu.sync_copy(acc_v, shared.at[pl.ds(s * _ACC, _ACC)])
    plsc.subcore_barrier()

    def _zero2(j, _):
        sum_v[pl.ds(j * 16, 16)] = z16
        return 0

    lax.fori_loop(0, _SL // 16, _zero2, 0)

    def _red(j, _):
        pltpu.sync_copy(shared.at[pl.ds(j * _ACC + s * _SL, _SL)], tmp_v)

        def _add(t, _):
            sum_v[pl.ds(t * 16, 16)] = (
                sum_v[pl.ds(t * 16, 16)] + tmp_v[pl.ds(t * 16, 16)])
            return 0

        lax.fori_loop(0, _SL // 16, _add, 0)
        return 0

    lax.fori_loop(0, 16, _red, 0)

    pltpu.sync_copy(sum_v, out_hbm.at[pl.ds(c * _ACC + s * _SL, _SL)])


def _sc_segsum(vals_flat, ids):
    f = pl.kernel(
        _segsum_body,
        mesh=plsc.VectorSubcoreMesh(core_axis_name="c", subcore_axis_name="s"),
        out_type=jax.ShapeDtypeStruct((2 * _ACC,), jnp.float32),
        compiler_params=pltpu.CompilerParams(needs_layout_passes=False),
        scratch_types=[
            pltpu.VMEM((K * _CH,), jnp.float32),
            pltpu.VMEM((_CH,), jnp.int32),
            pltpu.VMEM((_ACC,), jnp.float32),
            pltpu.VMEM((_SL,), jnp.float32),
            pltpu.VMEM((_SL,), jnp.float32),
            pltpu.VMEM_SHARED((16 * _ACC,), jnp.float32),
        ],
    )
    return f(vals_flat, ids)


def kernel(v, batch, W):
    vt = jnp.transpose(v, (2, 0, 1))
    mu_t = _tc_project(vt, W)
    out2 = _sc_segsum(mu_t.reshape(-1), batch.astype(jnp.int32))
    out2 = out2.reshape(2, _ACC)
    return (out2[0] + out2[1]).reshape(M, K)


# speedup vs baseline: 6.6959x; 1.0387x over previous
import functools

import jax
import jax.numpy as jnp
from jax import lax
from jax.experimental import pallas as pl
from jax.experimental.pallas import tpu as pltpu
from jax.experimental.pallas import tpu_sc as plsc

N = 100000
H = 128
M = 1024
K = 3

_NW = 32
_CH = 3200
_NPAD = _NW * _CH
_ACC = M * K
_SL = _ACC // 16
_TAIL = N - (_NW - 1) * _CH


_BN = 6400
_NB = _NPAD // _BN
_TCTAIL = N - (_NB - 1) * _BN


def _proj_body(v_ref, w_ref, o_ref):
    i = pl.program_id(0)
    for k in range(K):
        yk = lax.dot_general(
            w_ref[...], v_ref[k],
            (((1,), (1,)), ((), ())),
            preferred_element_type=jnp.float32,
        )

        @pl.when(i < _NB - 1)
        def _(yk=yk, k=k):
            o_ref[k:k + 1, :] = yk

        @pl.when(i == _NB - 1)
        def _(yk=yk, k=k):
            col = lax.broadcasted_iota(jnp.int32, (1, _BN), 1)
            o_ref[k:k + 1, :] = jnp.where(col < _TCTAIL, yk, 0.0)


def _tc_project(vt, w):
    return pl.pallas_call(
        _proj_body,
        grid=(_NB,),
        in_specs=[
            pl.BlockSpec((K, _BN, H), lambda i: (0, i, 0)),
            pl.BlockSpec((1, H), lambda i: (0, 0)),
        ],
        out_specs=pl.BlockSpec((K, _BN), lambda i: (0, i)),
        out_shape=jax.ShapeDtypeStruct((K, _NPAD), jnp.float32),
    )(vt, w)


def _segsum_body(vals_hbm, ids_hbm, out_hbm, vals_v, ids_v, acc_v, sum_v,
                 tmp_v, shared, dsem):
    c = lax.axis_index("c")
    s = lax.axis_index("s")
    wid = c * 16 + s
    base = wid * _CH

    vcopies = [
        pltpu.async_copy(vals_hbm.at[pl.ds(k * _NPAD + base, _CH)],
                         vals_v.at[pl.ds(k * _CH, _CH)], dsem)
        for k in range(K)
    ]

    @pl.when(wid < _NW - 1)
    def _():
        pltpu.async_copy(ids_hbm.at[pl.ds(base, _CH)], ids_v, dsem).wait()

    z16i = jnp.zeros((16,), jnp.int32)

    @pl.when(wid == _NW - 1)
    def _():
        pltpu.async_copy(ids_hbm.at[pl.ds((_NW - 1) * _CH, _TAIL)],
                         ids_v.at[pl.ds(0, _TAIL)], dsem).wait()

        def _zpad(j, _):
            ids_v[pl.ds(_TAIL + j * 16, 16)] = z16i
            return 0

        lax.fori_loop(0, (_CH - _TAIL) // 16, _zpad, 0)

    z16 = jnp.zeros((16,), jnp.float32)
    for cp in vcopies:
        cp.wait()

    def _zero(j, _):
        acc_v[pl.ds(j * 16, 16)] = z16
        return 0

    lax.fori_loop(0, _ACC // 16, _zero, 0)

    for k in range(K):
        def _accum(g, _, k=k):
            for u in range(4):
                o = g * 64 + u * 16
                idv = ids_v[pl.ds(o, 16)]
                vals = vals_v[pl.ds(k * _CH + o, 16)]
                plsc.addupdate_scatter(acc_v, [idv * 3 + k], vals)
            return 0

        lax.fori_loop(0, _CH // 64, _accum, 0)

    pltpu.sync_copy(acc_v, shared.at[pl.ds(s * _ACC, _ACC)])
    plsc.subcore_barrier()

    fetches = [
        pltpu.async_copy(shared.at[pl.ds(j * _ACC + s * _SL, _SL)],
                         tmp_v.at[pl.ds(j * _SL, _SL)], dsem)
        for j in range(16)
    ]
    for cp in fetches:
        cp.wait()

    def _add(t, _):
        def _add1(j, x):
            return x + tmp_v[pl.ds(j * _SL + t * 16, 16)]

        sum_v[pl.ds(t * 16, 16)] = lax.fori_loop(0, 16, _add1,
                                                 jnp.zeros((16,), jnp.float32))
        return 0

    lax.fori_loop(0, _SL // 16, _add, 0)

    pltpu.sync_copy(sum_v, out_hbm.at[pl.ds(c * _ACC + s * _SL, _SL)])


def _sc_segsum(vals_flat, ids):
    f = pl.kernel(
        _segsum_body,
        mesh=plsc.VectorSubcoreMesh(core_axis_name="c", subcore_axis_name="s"),
        out_type=jax.ShapeDtypeStruct((2 * _ACC,), jnp.float32),
        compiler_params=pltpu.CompilerParams(needs_layout_passes=False),
        scratch_types=[
            pltpu.VMEM((K * _CH,), jnp.float32),
            pltpu.VMEM((_CH,), jnp.int32),
            pltpu.VMEM((_ACC,), jnp.float32),
            pltpu.VMEM((_SL,), jnp.float32),
            pltpu.VMEM((16 * _SL,), jnp.float32),
            pltpu.VMEM_SHARED((16 * _ACC,), jnp.float32),
            pltpu.SemaphoreType.DMA,
        ],
    )
    return f(vals_flat, ids)


def kernel(v, batch, W):
    vt = jnp.transpose(v, (2, 0, 1))
    mu_t = _tc_project(vt, W)
    out2 = _sc_segsum(mu_t.reshape(-1), batch.astype(jnp.int32))
    out2 = out2.reshape(2, _ACC)
    return (out2[0] + out2[1]).reshape(M, K)
